# baseline (speedup 1.0000x reference)
import jax
import jax.numpy as jnp
from jax.experimental import pallas as pl
from jax.experimental.pallas import tpu as pltpu


def _body(keep_ref, x_ref, tok_ref, o_ref):
    t = pl.program_id(1)
    nb = pl.program_id(2)
    kk = keep_ref[t]
    rows = x_ref.shape[2]
    base = nb * rows
    row_ids = base + jax.lax.broadcasted_iota(jnp.int32, (rows, 1), 0)
    mask = row_ids < kk
    xv = x_ref[0, 0]
    tok = tok_ref[...]
    o_ref[0, 0] = jnp.where(mask, xv, tok[None, :])


def kernel(x, keep_k, mask_token):
    B, T, N, D = x.shape
    NB = 128
    grid = (B, T, N // NB)
    return pl.pallas_call(
        _body,
        grid_spec=pltpu.PrefetchScalarGridSpec(
            num_scalar_prefetch=1,
            grid=grid,
            in_specs=[
                pl.BlockSpec((1, 1, NB, D), lambda b, t, n, kref: (b, t, n, 0)),
                pl.BlockSpec((D,), lambda b, t, n, kref: (0,)),
            ],
            out_specs=pl.BlockSpec((1, 1, NB, D), lambda b, t, n, kref: (b, t, n, 0)),
        ),
        out_shape=jax.ShapeDtypeStruct(x.shape, x.dtype),
    )(keep_k.astype(jnp.int32), x, mask_token)


# speedup vs baseline: 2.2113x; 2.2113x over previous
import jax
import jax.numpy as jnp
from jax.experimental import pallas as pl
from jax.experimental.pallas import tpu as pltpu


def _body(keep_ref, x_ref, tok_ref, o_ref):
    t = pl.program_id(1)
    nb = pl.program_id(2)
    kk = keep_ref[t]
    rows = x_ref.shape[2]
    base = nb * rows
    row_ids = base + jax.lax.broadcasted_iota(jnp.int32, (rows, 1), 0)
    mask = row_ids < kk
    xv = x_ref[0, 0]
    tok = tok_ref[...]
    o_ref[0, 0] = jnp.where(mask, xv, tok[None, :])


def kernel(x, keep_k, mask_token):
    B, T, N, D = x.shape
    NB = 512
    grid = (B, T, N // NB)
    return pl.pallas_call(
        _body,
        grid_spec=pltpu.PrefetchScalarGridSpec(
            num_scalar_prefetch=1,
            grid=grid,
            in_specs=[
                pl.BlockSpec((1, 1, NB, D), lambda b, t, n, kref: (b, t, n, 0)),
                pl.BlockSpec((D,), lambda b, t, n, kref: (0,)),
            ],
            out_specs=pl.BlockSpec((1, 1, NB, D), lambda b, t, n, kref: (b, t, n, 0)),
        ),
        out_shape=jax.ShapeDtypeStruct(x.shape, x.dtype),
    )(keep_k.astype(jnp.int32), x, mask_token)


# speedup vs baseline: 2.6276x; 1.1883x over previous
import jax
import jax.numpy as jnp
from jax.experimental import pallas as pl
from jax.experimental.pallas import tpu as pltpu


def _body(keep_ref, x_ref, tok_ref, o_ref):
    t = pl.program_id(1)
    nb = pl.program_id(2)
    kk = keep_ref[t]
    rows = x_ref.shape[2]
    base = nb * rows
    row_ids = base + jax.lax.broadcasted_iota(jnp.int32, (rows, 1), 0)
    mask = row_ids < kk
    xv = x_ref[0, 0]
    tok = tok_ref[...]
    o_ref[0, 0] = jnp.where(mask, xv, tok[None, :])


def kernel(x, keep_k, mask_token):
    B, T, N, D = x.shape
    NB = 1024
    grid = (B, T, N // NB)
    return pl.pallas_call(
        _body,
        grid_spec=pltpu.PrefetchScalarGridSpec(
            num_scalar_prefetch=1,
            grid=grid,
            in_specs=[
                pl.BlockSpec((1, 1, NB, D), lambda b, t, n, kref: (b, t, n, 0)),
                pl.BlockSpec((D,), lambda b, t, n, kref: (0,)),
            ],
            out_specs=pl.BlockSpec((1, 1, NB, D), lambda b, t, n, kref: (b, t, n, 0)),
        ),
        out_shape=jax.ShapeDtypeStruct(x.shape, x.dtype),
    )(keep_k.astype(jnp.int32), x, mask_token)
